# baseline (speedup 1.0000x reference)
import functools

import jax
import jax.numpy as jnp
from jax import lax
from jax.experimental import pallas as pl
from jax.experimental.pallas import tpu as pltpu
from jax.experimental.pallas import tpu_sc as plsc

BATCH = 16384
SEQ_LEN = 7
NUM_WORKERS = 32
COLS_W = BATCH // NUM_WORKERS
ROW = 128
NCHUNK = COLS_W // ROW


PER_W = SEQ_LEN * COLS_W
CHUNKS = [(r, j) for r in range(SEQ_LEN) for j in range(NCHUNK)]
GSIZE = 4
GROUPS = [CHUNKS[i:i + GSIZE] for i in range(0, len(CHUNKS), GSIZE)]


def _gather_kernel(idx_hbm, table_hbm, out_hbm, idx_v, res_v, sem_s, sem_g, sem_w):
    wid = lax.axis_index("s") * 2 + lax.axis_index("c")
    base = wid * COLS_W

    def refs(r):
        hbm_idx = (pl.ds(r, 1), pl.ds(base, COLS_W))
        vcol = pl.ds(r * COLS_W, COLS_W)
        return hbm_idx, vcol

    def stage_issue(r, c):
        hbm_idx, vcol = refs(r)
        pltpu.async_copy(idx_hbm.at[hbm_idx], idx_v.at[:, vcol], sem_s)
        return c

    def stage_drain(r, c):
        hbm_idx, vcol = refs(r)
        pltpu.make_async_copy(idx_hbm.at[hbm_idx], idx_v.at[:, vcol], sem_s).wait()
        return c

    def gather_issue(r, c):
        _, vcol = refs(r)
        pltpu.async_copy(table_hbm.at[idx_v.at[0, vcol]], res_v.at[0, vcol], sem_g)
        return c

    def gather_drain(r, c):
        _, vcol = refs(r)
        pltpu.make_async_copy(
            table_hbm.at[idx_v.at[0, vcol]], res_v.at[0, vcol], sem_g
        ).wait()
        return c

    def write_issue(r, c):
        hbm_idx, vcol = refs(r)
        pltpu.async_copy(res_v.at[:, vcol], out_hbm.at[hbm_idx], sem_w)
        return c

    def write_drain(r, c):
        hbm_idx, vcol = refs(r)
        pltpu.make_async_copy(res_v.at[:, vcol], out_hbm.at[hbm_idx], sem_w).wait()
        return c

    lax.fori_loop(0, SEQ_LEN, stage_issue, 0)
    lax.fori_loop(0, SEQ_LEN, stage_drain, 0)
    lax.fori_loop(0, SEQ_LEN, gather_issue, 0)
    lax.fori_loop(0, SEQ_LEN, gather_drain, 0)
    lax.fori_loop(0, SEQ_LEN, write_issue, 0)
    lax.fori_loop(0, SEQ_LEN, write_drain, 0)


@jax.jit
def _run(idx_t, table):
    mesh = plsc.VectorSubcoreMesh(core_axis_name="c", subcore_axis_name="s")
    fn = functools.partial(
        pl.kernel,
        out_type=jax.ShapeDtypeStruct((SEQ_LEN, BATCH), jnp.int32),
        mesh=mesh,
        scratch_types=[
            pltpu.VMEM((1, PER_W), jnp.int32),
            pltpu.VMEM((1, PER_W), jnp.int32),
            pltpu.SemaphoreType.DMA,
            pltpu.SemaphoreType.DMA,
            pltpu.SemaphoreType.DMA,
        ],
    )(_gather_kernel)
    return fn(idx_t, table)


def kernel(faked_id, lookup_table):
    out_t = _run(faked_id.T, lookup_table)
    return out_t.T


# speedup vs baseline: 1.0375x; 1.0375x over previous
import functools

import jax
import jax.numpy as jnp
from jax import lax
from jax.experimental import pallas as pl
from jax.experimental.pallas import tpu as pltpu
from jax.experimental.pallas import tpu_sc as plsc

BATCH = 16384
SEQ_LEN = 7
NUM_WORKERS = 32
COLS_W = BATCH // NUM_WORKERS
ROW = 128
NCHUNK = COLS_W // ROW


PER_W = SEQ_LEN * COLS_W
CHUNKS = [(r, j) for r in range(SEQ_LEN) for j in range(NCHUNK)]
GSIZE = 4
GROUPS = [CHUNKS[i:i + GSIZE] for i in range(0, len(CHUNKS), GSIZE)]


def _gather_kernel(idx_hbm, table_hbm, out_hbm, idx_v, res_v, sem_s, sem_g, sem_w):
    wid = lax.axis_index("s") * 2 + lax.axis_index("c")
    base = wid * COLS_W

    def refs(r):
        hbm_idx = (pl.ds(r, 1), pl.ds(base, COLS_W))
        vcol = pl.ds(r * COLS_W, COLS_W)
        return hbm_idx, vcol

    def stage_issue(r, c):
        hbm_idx, vcol = refs(r)
        pltpu.async_copy(idx_hbm.at[hbm_idx], idx_v.at[:, vcol], sem_s.at[r])
        return c

    def gather_issue(r, c):
        hbm_idx, vcol = refs(r)
        pltpu.make_async_copy(
            idx_hbm.at[hbm_idx], idx_v.at[:, vcol], sem_s.at[r]
        ).wait()
        pltpu.async_copy(
            table_hbm.at[idx_v.at[0, vcol]], res_v.at[0, vcol], sem_g.at[r]
        )
        return c

    def write_issue(r, c):
        hbm_idx, vcol = refs(r)
        pltpu.make_async_copy(
            table_hbm.at[idx_v.at[0, vcol]], res_v.at[0, vcol], sem_g.at[r]
        ).wait()
        pltpu.async_copy(res_v.at[:, vcol], out_hbm.at[hbm_idx], sem_w)
        return c

    def write_drain(r, c):
        hbm_idx, vcol = refs(r)
        pltpu.make_async_copy(res_v.at[:, vcol], out_hbm.at[hbm_idx], sem_w).wait()
        return c

    lax.fori_loop(0, SEQ_LEN, stage_issue, 0)
    lax.fori_loop(0, SEQ_LEN, gather_issue, 0)
    lax.fori_loop(0, SEQ_LEN, write_issue, 0)
    lax.fori_loop(0, SEQ_LEN, write_drain, 0)


@jax.jit
def _run(idx_t, table):
    mesh = plsc.VectorSubcoreMesh(core_axis_name="c", subcore_axis_name="s")
    fn = functools.partial(
        pl.kernel,
        out_type=jax.ShapeDtypeStruct((SEQ_LEN, BATCH), jnp.int32),
        mesh=mesh,
        scratch_types=[
            pltpu.VMEM((1, PER_W), jnp.int32),
            pltpu.VMEM((1, PER_W), jnp.int32),
            pltpu.SemaphoreType.DMA((SEQ_LEN,)),
            pltpu.SemaphoreType.DMA((SEQ_LEN,)),
            pltpu.SemaphoreType.DMA,
        ],
    )(_gather_kernel)
    return fn(idx_t, table)


def kernel(faked_id, lookup_table):
    out_t = _run(faked_id.T, lookup_table)
    return out_t.T
